# baseline (speedup 1.0000x reference)
import functools

import jax
import jax.numpy as jnp
from jax import lax
from jax.experimental import pallas as pl
from jax.experimental.pallas import tpu as pltpu
from jax.experimental.pallas import tpu_sc as plsc

NN = 10000
NPAD = 10240
TT = 12
HH = 128
EE = 320000
NHEADS = 4
DH = HH // NHEADS

NSUB = 16
NCORE = 2
SLAB = NPAD // NSUB
CH = 128
CPW = 79
EPW = CPW * CH
EPAD = NCORE * NSUB * EPW
NCOLS = TT + 1
PASSES = ((0, 1, 2), (3, 4, 5), (6, 7, 8), (9, 10, 11), (12,))
MAXP = 3


def _sc_deg_body(dst_hbm, out_hbm, dstbuf, acc, redbuf, shared):
    c = lax.axis_index("c")
    s = lax.axis_index("s")
    w = c * NSUB + s
    ones16 = jnp.ones((16,), jnp.float32)
    zero16 = jnp.zeros((16,), jnp.float32)

    pltpu.sync_copy(dst_hbm.at[pl.ds(w * EPW, EPW)], dstbuf)

    def zb(i, cr):
        acc[pl.ds(i * 16, 16)] = zero16
        return cr

    lax.fori_loop(0, NPAD // 16, zb, 0)

    def eb(o, cr):
        for u2 in range(CH // 16):
            di = dstbuf[pl.ds(o * CH + u2 * 16, 16)]
            plsc.addupdate_scatter(acc, [di], ones16)
        return cr

    lax.fori_loop(0, CPW, eb, 0)

    pltpu.sync_copy(acc, shared.at[s])
    plsc.subcore_barrier()
    base = s * SLAB
    for half in range(2):
        pltpu.sync_copy(shared.at[pl.ds(half * 8, 8), pl.ds(base, SLAB)],
                        redbuf)

        def rb(g, cr, _half=half):
            v = redbuf[0, pl.ds(g * 16, 16)]
            for rr in range(1, 8):
                v = v + redbuf[rr, pl.ds(g * 16, 16)]
            o = base + g * 16
            if _half:
                acc[pl.ds(o, 16)] = acc[pl.ds(o, 16)] + v
            else:
                acc[pl.ds(o, 16)] = v
            return cr

        lax.fori_loop(0, SLAB // 16, rb, 0)
    pltpu.sync_copy(acc.at[pl.ds(base, SLAB)], out_hbm.at[c, pl.ds(base, SLAB)])


def _sc_gs_body(src_hbm, dst_hbm, ztt_hbm, out_hbm, srcbuf, dstbuf,
                tb0, tb1, tb2, ac0, ac1, ac2,
                redbuf, shared):
    c = lax.axis_index("c")
    s = lax.axis_index("s")
    w = c * NSUB + s
    zero16 = jnp.zeros((16,), jnp.float32)
    tables = (tb0, tb1, tb2)
    accs = (ac0, ac1, ac2)

    pltpu.sync_copy(src_hbm.at[pl.ds(w * EPW, EPW)], srcbuf)
    pltpu.sync_copy(dst_hbm.at[pl.ds(w * EPW, EPW)], dstbuf)
    base = s * SLAB

    for cols in PASSES:
        ncol = len(cols)

        def zb(i, cr, _ncol=ncol):
            for ci in range(_ncol):
                accs[ci][pl.ds(i * 16, 16)] = zero16
            return cr

        lax.fori_loop(0, NPAD // 16, zb, 0)
        for ci, col in enumerate(cols):
            pltpu.sync_copy(ztt_hbm.at[col], tables[ci])

        def eb(o, cr, _ncol=ncol):
            for u2 in range(CH // 16):
                off = o * CH + u2 * 16
                si = srcbuf[pl.ds(off, 16)]
                di = dstbuf[pl.ds(off, 16)]
                for ci in range(_ncol):
                    vals = plsc.load_gather(tables[ci], [si])
                    plsc.addupdate_scatter(accs[ci], [di], vals)
            return cr

        lax.fori_loop(0, CPW, eb, 0)

        for ci in range(ncol):
            pltpu.sync_copy(accs[ci], shared.at[ci, s])
        plsc.subcore_barrier()
        for ci, col in enumerate(cols):
            for half in range(2):
                pltpu.sync_copy(
                    shared.at[ci, pl.ds(half * 8, 8), pl.ds(base, SLAB)],
                    redbuf)

                def rb(g, cr, _ci=ci, _half=half):
                    v = redbuf[0, pl.ds(g * 16, 16)]
                    for rr in range(1, 8):
                        v = v + redbuf[rr, pl.ds(g * 16, 16)]
                    o = base + g * 16
                    if _half:
                        accs[_ci][pl.ds(o, 16)] = accs[_ci][pl.ds(o, 16)] + v
                    else:
                        accs[_ci][pl.ds(o, 16)] = v
                    return cr

                lax.fori_loop(0, SLAB // 16, rb, 0)
            pltpu.sync_copy(accs[ci].at[pl.ds(base, SLAB)],
                            out_hbm.at[c, col, pl.ds(base, SLAB)])
        plsc.subcore_barrier()


@functools.cache
def _sc_kernels():
    mesh = plsc.VectorSubcoreMesh(core_axis_name="c", subcore_axis_name="s",
                                  num_cores=NCORE, num_subcores=NSUB)
    params = pltpu.CompilerParams(needs_layout_passes=False,
                                  use_tc_tiling_on_sc=False)
    sc_deg = pl.kernel(
        _sc_deg_body,
        out_type=jax.ShapeDtypeStruct((NCORE, NPAD), jnp.float32),
        mesh=mesh,
        compiler_params=params,
        scratch_types=[
            pltpu.VMEM((EPW,), jnp.int32),
            pltpu.VMEM((NPAD,), jnp.float32),
            pltpu.VMEM((8, SLAB), jnp.float32),
            pltpu.VMEM_SHARED((NSUB, NPAD), jnp.float32),
        ],
    )
    sc_gs = pl.kernel(
        _sc_gs_body,
        out_type=jax.ShapeDtypeStruct((NCORE, NCOLS, NPAD), jnp.float32),
        mesh=mesh,
        compiler_params=params,
        scratch_types=(
            [pltpu.VMEM((EPW,), jnp.int32)] * 2
            + [pltpu.VMEM((NPAD,), jnp.float32)] * (2 * MAXP)
            + [pltpu.VMEM((8, SLAB), jnp.float32),
               pltpu.VMEM_SHARED((MAXP, NSUB, NPAD), jnp.float32)]
        ),
    )
    return sc_deg, sc_gs


def _prep_body(s16_ref, d0_ref, d1_ref, ztt_ref):
    deg = d0_ref[...] + d1_ref[...] + 1.0
    dinv = lax.rsqrt(deg)
    sel = (lax.broadcasted_iota(jnp.int32, (16, 1), 0) == TT).astype(
        jnp.float32)
    mm = lambda x, w_: lax.dot_general(
        x, w_, (((1,), (0,)), ((), ())), preferred_element_type=jnp.float32)
    ztt_ref[...] = s16_ref[...] * dinv + mm(sel, dinv)


def _tc_prep(s16, d0, d1, bn=1024):
    return pl.pallas_call(
        _prep_body,
        grid=(NPAD // bn,),
        in_specs=[
            pl.BlockSpec((16, bn), lambda i: (0, i)),
            pl.BlockSpec((1, bn), lambda i: (0, i)),
            pl.BlockSpec((1, bn), lambda i: (0, i)),
        ],
        out_specs=pl.BlockSpec((16, bn), lambda i: (0, i)),
        out_shape=jax.ShapeDtypeStruct((16, NPAD), jnp.float32),
    )(s16, d0, d1)


def _main_body(qp_ref, zt_ref, Wgcn, Wih, Whh, Wip, Wop, Wp1, Wp2,
               we, be, bg, bih, bhh, bip, bop, lng, lnb, bp1, bp2, out_ref):
    f32 = jnp.float32
    dot = lambda x, w_: lax.dot_general(
        x, w_, (((1,), (1,)), ((), ())), preferred_element_type=f32)
    mm = lambda x, w_: lax.dot_general(
        x, w_, (((1,), (0,)), ((), ())), preferred_element_type=f32)
    ones16 = jnp.ones((1, 16), f32)
    bc16 = lambda col: mm(col, ones16)

    ztb = zt_ref[...]
    dinv = ztb[:, 12:13]
    acols = bc16(dinv) * (qp_ref[0] + qp_ref[1] + ztb)
    dd = acols[:, 12:13]
    a_last = acols[:, 11:12]
    lane16 = lax.broadcasted_iota(jnp.int32, (1, 16), 1)
    tmask = (lane16 < TT).astype(f32)
    a16 = acols * tmask

    we_, be_, bg_ = we[...], be[...], bg[...]
    u = dot(we_, Wgcn[...])
    cvec = dot(be_, Wgcn[...])
    ui = dot(u, Wih[...])
    ci = dot(cvec, Wih[...])
    bi0 = dot(bg_, Wih[...]) + bih[...]

    bn = ztb.shape[0]
    h = jnp.zeros((bn, HH), f32)
    gi_base = mm(dd, ci) + bi0
    bhh_r = bhh[...]
    for t in range(TT):
        gi = mm(acols[:, t:t + 1], ui) + gi_base
        gh = dot(h, Whh[...]) + bhh_r
        r = jax.nn.sigmoid(gi[:, 0:HH] + gh[:, 0:HH])
        z = jax.nn.sigmoid(gi[:, HH:2 * HH] + gh[:, HH:2 * HH])
        ng = jnp.tanh(gi[:, 2 * HH:] + r * gh[:, 2 * HH:])
        h = (1.0 - z) * ng + z * h

    pu = dot(u, Wip[...])
    pc = dot(cvec, Wip[...])
    pb = dot(bg_, Wip[...]) + bip[...]
    scl = 1.0 / (DH ** 0.5)

    puv = pu[:, 2 * HH:]
    pcv = pc[:, 2 * HH:]
    pbv = pb[:, 2 * HH:]
    cv = dot(pcv, Wop[...])
    bv = dot(pbv, Wop[...]) + bop[...]

    y = mm(a_last, u) + mm(dd, cvec + cv) + (bg_ + bv)
    negbig = (lane16 >= TT).astype(f32) * (-1e30)
    for hd in range(NHEADS):
        o = hd * DH
        puq = pu[:, o:o + DH]; puk = pu[:, HH + o:HH + o + DH]
        pcq = pc[:, o:o + DH]; pck = pc[:, HH + o:HH + o + DH]
        pbq = pb[:, o:o + DH]; pbk = pb[:, HH + o:HH + o + DH]
        d11 = lambda x, yv: jnp.sum(x * yv)
        c_qk = d11(puq, puk)
        c_qB = d11(puq, pck); c_qb = d11(puq, pbk)
        c_Ak = d11(pcq, puk); c_ak = d11(pbq, puk)
        c_AA = d11(pcq, pck)
        c_Ab = d11(pcq, pbk) + d11(pbq, pck)
        c_bb = d11(pbq, pbk)
        f1 = (a_last * c_qk + dd * c_Ak + c_ak) * scl
        f0 = (a_last * (dd * c_qB + c_qb)
              + dd * dd * c_AA + dd * c_Ab + c_bb) * scl
        scores = a16 * bc16(f1) + bc16(f0) + negbig
        m = jnp.max(scores, axis=1, keepdims=True)
        e = jnp.exp(scores - bc16(m))
        recip = 1.0 / jnp.sum(e, axis=1, keepdims=True)
        att = e * bc16(recip)
        wh = jnp.sum(att * a16, axis=1, keepdims=True)
        eh = dot(puv[:, o:o + DH], Wop[:, o:o + DH])
        y = y + mm(wh, eh)

    ones128 = jnp.ones((1, HH), f32)
    mu = jnp.mean(y, axis=1, keepdims=True)
    yc = y - mm(mu, ones128)
    var = jnp.mean(yc * yc, axis=1, keepdims=True)
    irs = mm(lax.rsqrt(var + 1e-5), ones128)
    gt = yc * irs * lng[...] + lnb[...]

    comb = jnp.concatenate([h, gt], axis=1)
    hm = jax.nn.relu(dot(comb, Wp1[...]) + bp1[...])
    o8 = dot(hm, Wp2[...])
    out_ref[...] = jax.nn.sigmoid(o8[:, 0:1] + bp2[0, 0])


def _tc_main(qp, zt, weights, bn=1024):
    nb = NPAD // bn
    full = lambda shape: pl.BlockSpec(shape, lambda i: tuple(0 for _ in shape))
    in_specs = [
        pl.BlockSpec((NCORE, bn, 16), lambda i: (0, i, 0)),
        pl.BlockSpec((bn, 16), lambda i: (i, 0)),
    ] + [full(w.shape) for w in weights]
    return pl.pallas_call(
        _main_body,
        grid=(nb,),
        in_specs=in_specs,
        out_specs=pl.BlockSpec((bn, 1), lambda i: (i, 0)),
        out_shape=jax.ShapeDtypeStruct((NPAD, 1), jnp.float32),
    )(qp, zt, *weights)


def kernel(snapshots, edge_index, W_embed, b_embed, W_gcn, b_gcn, W_ih, W_hh,
           b_ih, b_hh, W_in_proj, b_in_proj, W_out_proj, b_out_proj, ln_g,
           ln_b, W_p1, b_p1, W_p2, b_p2):
    f32 = jnp.float32
    src = edge_index[0]
    dst = edge_index[1]
    sink = NN + (jnp.arange(EPAD - EE, dtype=jnp.int32) % (NPAD - NN))
    src_p = jnp.concatenate([src, sink])
    dst_p = jnp.concatenate([dst, sink])

    sc_deg, sc_gs = _sc_kernels()
    degp = sc_deg(dst_p)
    s16 = jnp.zeros((16, NPAD), f32).at[0:TT, :NN].set(snapshots)
    ztt = _tc_prep(s16, degp[0:1], degp[1:2])
    qt = sc_gs(src_p, dst_p, ztt)

    qp = jnp.zeros((NCORE, NPAD, 16), f32).at[:, :, 0:NCOLS].set(
        jnp.transpose(qt, (0, 2, 1)))
    zt = jnp.transpose(ztt, (1, 0))

    r1 = lambda v: v.reshape(1, -1)
    W_p2_pad = jnp.zeros((8, HH), f32).at[0:1, :].set(W_p2)
    weights = (W_gcn, W_ih, W_hh, W_in_proj, W_out_proj, W_p1, W_p2_pad,
               r1(W_embed), r1(b_embed), r1(b_gcn), r1(b_ih), r1(b_hh),
               r1(b_in_proj), r1(b_out_proj), r1(ln_g), r1(ln_b), r1(b_p1),
               r1(b_p2))
    out = _tc_main(qp, zt, weights)
    return out[:NN, 0]


# speedup vs baseline: 1.0585x; 1.0585x over previous
import functools

import jax
import jax.numpy as jnp
from jax import lax
from jax.experimental import pallas as pl
from jax.experimental.pallas import tpu as pltpu
from jax.experimental.pallas import tpu_sc as plsc

NN = 10000
NPAD = 10240
TT = 12
HH = 128
EE = 320000
NHEADS = 4
DH = HH // NHEADS

NSUB = 16
NCORE = 2
SLAB = NPAD // NSUB
CH = 128
CPW = 79
EPW = CPW * CH
EPAD = NCORE * NSUB * EPW
NCOLS = TT + 1
PASSES = ((0, 1, 2, 3), (4, 5, 6, 7), (8, 9, 10, 11), (12,))
MAXP = 4


def _sc_deg_body(dst_hbm, out_hbm, dstbuf, acc, redbuf, shared):
    c = lax.axis_index("c")
    s = lax.axis_index("s")
    w = c * NSUB + s
    ones16 = jnp.ones((16,), jnp.float32)
    zero16 = jnp.zeros((16,), jnp.float32)

    pltpu.sync_copy(dst_hbm.at[pl.ds(w * EPW, EPW)], dstbuf)

    def zb(i, cr):
        acc[pl.ds(i * 16, 16)] = zero16
        return cr

    lax.fori_loop(0, NPAD // 16, zb, 0)

    def eb(o, cr):
        for u2 in range(CH // 16):
            di = dstbuf[pl.ds(o * CH + u2 * 16, 16)]
            plsc.addupdate_scatter(acc, [di], ones16)
        return cr

    lax.fori_loop(0, CPW, eb, 0)

    pltpu.sync_copy(acc, shared.at[s])
    plsc.subcore_barrier()
    base = s * SLAB
    for half in range(2):
        pltpu.sync_copy(shared.at[pl.ds(half * 8, 8), pl.ds(base, SLAB)],
                        redbuf)

        def rb(g, cr, _half=half):
            v = redbuf[0, pl.ds(g * 16, 16)]
            for rr in range(1, 8):
                v = v + redbuf[rr, pl.ds(g * 16, 16)]
            o = base + g * 16
            if _half:
                acc[pl.ds(o, 16)] = acc[pl.ds(o, 16)] + v
            else:
                acc[pl.ds(o, 16)] = v
            return cr

        lax.fori_loop(0, SLAB // 16, rb, 0)
    pltpu.sync_copy(acc.at[pl.ds(base, SLAB)], out_hbm.at[c, pl.ds(base, SLAB)])


def _sc_gs_body(src_hbm, dst_hbm, ztt_hbm, out_hbm, srcbuf, dstbuf,
                tb0, tb1, tb2, tb3, ac0, ac1, ac2, ac3,
                redbuf, shared):
    c = lax.axis_index("c")
    s = lax.axis_index("s")
    w = c * NSUB + s
    zero16 = jnp.zeros((16,), jnp.float32)
    tables = (tb0, tb1, tb2, tb3)
    accs = (ac0, ac1, ac2, ac3)

    pltpu.sync_copy(src_hbm.at[pl.ds(w * EPW, EPW)], srcbuf)
    pltpu.sync_copy(dst_hbm.at[pl.ds(w * EPW, EPW)], dstbuf)
    base = s * SLAB

    for cols in PASSES:
        ncol = len(cols)

        def zb(i, cr, _ncol=ncol):
            for ci in range(_ncol):
                accs[ci][pl.ds(i * 16, 16)] = zero16
            return cr

        lax.fori_loop(0, NPAD // 16, zb, 0)
        for ci, col in enumerate(cols):
            pltpu.sync_copy(ztt_hbm.at[col], tables[ci])

        def eb(o, cr, _ncol=ncol):
            for u2 in range(CH // 16):
                off = o * CH + u2 * 16
                si = srcbuf[pl.ds(off, 16)]
                di = dstbuf[pl.ds(off, 16)]
                for ci in range(_ncol):
                    vals = plsc.load_gather(tables[ci], [si])
                    plsc.addupdate_scatter(accs[ci], [di], vals)
            return cr

        lax.fori_loop(0, CPW, eb, 0)

        for ci, col in enumerate(cols):
            pltpu.sync_copy(accs[ci], shared.at[s])
            plsc.subcore_barrier()
            for half in range(2):
                pltpu.sync_copy(
                    shared.at[pl.ds(half * 8, 8), pl.ds(base, SLAB)],
                    redbuf)

                def rb(g, cr, _ci=ci, _half=half):
                    v = redbuf[0, pl.ds(g * 16, 16)]
                    for rr in range(1, 8):
                        v = v + redbuf[rr, pl.ds(g * 16, 16)]
                    o = base + g * 16
                    if _half:
                        accs[_ci][pl.ds(o, 16)] = accs[_ci][pl.ds(o, 16)] + v
                    else:
                        accs[_ci][pl.ds(o, 16)] = v
                    return cr

                lax.fori_loop(0, SLAB // 16, rb, 0)
            pltpu.sync_copy(accs[ci].at[pl.ds(base, SLAB)],
                            out_hbm.at[c, col, pl.ds(base, SLAB)])
            plsc.subcore_barrier()


@functools.cache
def _sc_kernels():
    mesh = plsc.VectorSubcoreMesh(core_axis_name="c", subcore_axis_name="s",
                                  num_cores=NCORE, num_subcores=NSUB)
    params = pltpu.CompilerParams(needs_layout_passes=False,
                                  use_tc_tiling_on_sc=False)
    sc_deg = pl.kernel(
        _sc_deg_body,
        out_type=jax.ShapeDtypeStruct((NCORE, NPAD), jnp.float32),
        mesh=mesh,
        compiler_params=params,
        scratch_types=[
            pltpu.VMEM((EPW,), jnp.int32),
            pltpu.VMEM((NPAD,), jnp.float32),
            pltpu.VMEM((8, SLAB), jnp.float32),
            pltpu.VMEM_SHARED((NSUB, NPAD), jnp.float32),
        ],
    )
    sc_gs = pl.kernel(
        _sc_gs_body,
        out_type=jax.ShapeDtypeStruct((NCORE, NCOLS, NPAD), jnp.float32),
        mesh=mesh,
        compiler_params=params,
        scratch_types=(
            [pltpu.VMEM((EPW,), jnp.int32)] * 2
            + [pltpu.VMEM((NPAD,), jnp.float32)] * (2 * MAXP)
            + [pltpu.VMEM((8, SLAB), jnp.float32),
               pltpu.VMEM_SHARED((NSUB, NPAD), jnp.float32)]
        ),
    )
    return sc_deg, sc_gs


def _prep_body(s16_ref, d0_ref, d1_ref, ztt_ref):
    deg = d0_ref[...] + d1_ref[...] + 1.0
    dinv = lax.rsqrt(deg)
    sel = (lax.broadcasted_iota(jnp.int32, (16, 1), 0) == TT).astype(
        jnp.float32)
    mm = lambda x, w_: lax.dot_general(
        x, w_, (((1,), (0,)), ((), ())), preferred_element_type=jnp.float32)
    ztt_ref[...] = s16_ref[...] * dinv + mm(sel, dinv)


def _tc_prep(s16, d0, d1, bn=1024):
    return pl.pallas_call(
        _prep_body,
        grid=(NPAD // bn,),
        in_specs=[
            pl.BlockSpec((16, bn), lambda i: (0, i)),
            pl.BlockSpec((1, bn), lambda i: (0, i)),
            pl.BlockSpec((1, bn), lambda i: (0, i)),
        ],
        out_specs=pl.BlockSpec((16, bn), lambda i: (0, i)),
        out_shape=jax.ShapeDtypeStruct((16, NPAD), jnp.float32),
    )(s16, d0, d1)


def _main_body(qt_ref, ztt_ref, Wgcn, Wih, Whh, Wip, Wop, Wp1, Wp2,
               we, be, bg, bih, bhh, bip, bop, lng, lnb, bp1, bp2, out_ref):
    f32 = jnp.float32
    dot = lambda x, w_: lax.dot_general(
        x, w_, (((1,), (1,)), ((), ())), preferred_element_type=f32)
    mm = lambda x, w_: lax.dot_general(
        x, w_, (((1,), (0,)), ((), ())), preferred_element_type=f32)
    ones16 = jnp.ones((1, 16), f32)
    bc16 = lambda col: mm(col, ones16)

    zttb = ztt_ref[...]
    bn = zttb.shape[1]
    dinv_row = zttb[12:13, :]
    qpad = jnp.concatenate(
        [qt_ref[0] + qt_ref[1], jnp.zeros((16 - NCOLS, bn), f32)], axis=0)
    acols = jnp.transpose(dinv_row * (qpad + zttb), (1, 0))
    dd = acols[:, 12:13]
    a_last = acols[:, 11:12]
    lane16 = lax.broadcasted_iota(jnp.int32, (1, 16), 1)
    tmask = (lane16 < TT).astype(f32)
    a16 = acols * tmask

    we_, be_, bg_ = we[...], be[...], bg[...]
    u = dot(we_, Wgcn[...])
    cvec = dot(be_, Wgcn[...])
    ui = dot(u, Wih[...])
    ci = dot(cvec, Wih[...])
    bi0 = dot(bg_, Wih[...]) + bih[...]

    h = jnp.zeros((bn, HH), f32)
    gi_base = mm(dd, ci) + bi0
    bhh_r = bhh[...]
    for t in range(TT):
        gi = mm(acols[:, t:t + 1], ui) + gi_base
        gh = dot(h, Whh[...]) + bhh_r
        r = jax.nn.sigmoid(gi[:, 0:HH] + gh[:, 0:HH])
        z = jax.nn.sigmoid(gi[:, HH:2 * HH] + gh[:, HH:2 * HH])
        ng = jnp.tanh(gi[:, 2 * HH:] + r * gh[:, 2 * HH:])
        h = (1.0 - z) * ng + z * h

    pu = dot(u, Wip[...])
    pc = dot(cvec, Wip[...])
    pb = dot(bg_, Wip[...]) + bip[...]
    scl = 1.0 / (DH ** 0.5)

    puv = pu[:, 2 * HH:]
    pcv = pc[:, 2 * HH:]
    pbv = pb[:, 2 * HH:]
    cv = dot(pcv, Wop[...])
    bv = dot(pbv, Wop[...]) + bop[...]

    y = mm(a_last, u) + mm(dd, cvec + cv) + (bg_ + bv)
    negbig = (lane16 >= TT).astype(f32) * (-1e30)
    for hd in range(NHEADS):
        o = hd * DH
        puq = pu[:, o:o + DH]; puk = pu[:, HH + o:HH + o + DH]
        pcq = pc[:, o:o + DH]; pck = pc[:, HH + o:HH + o + DH]
        pbq = pb[:, o:o + DH]; pbk = pb[:, HH + o:HH + o + DH]
        d11 = lambda x, yv: jnp.sum(x * yv)
        c_qk = d11(puq, puk)
        c_qB = d11(puq, pck); c_qb = d11(puq, pbk)
        c_Ak = d11(pcq, puk); c_ak = d11(pbq, puk)
        c_AA = d11(pcq, pck)
        c_Ab = d11(pcq, pbk) + d11(pbq, pck)
        c_bb = d11(pbq, pbk)
        f1 = (a_last * c_qk + dd * c_Ak + c_ak) * scl
        f0 = (a_last * (dd * c_qB + c_qb)
              + dd * dd * c_AA + dd * c_Ab + c_bb) * scl
        scores = a16 * bc16(f1) + bc16(f0) + negbig
        m = jnp.max(scores, axis=1, keepdims=True)
        e = jnp.exp(scores - bc16(m))
        recip = 1.0 / jnp.sum(e, axis=1, keepdims=True)
        att = e * bc16(recip)
        wh = jnp.sum(att * a16, axis=1, keepdims=True)
        eh = dot(puv[:, o:o + DH], Wop[:, o:o + DH])
        y = y + mm(wh, eh)

    ones128 = jnp.ones((1, HH), f32)
    mu = jnp.mean(y, axis=1, keepdims=True)
    yc = y - mm(mu, ones128)
    var = jnp.mean(yc * yc, axis=1, keepdims=True)
    irs = mm(lax.rsqrt(var + 1e-5), ones128)
    gt = yc * irs * lng[...] + lnb[...]

    comb = jnp.concatenate([h, gt], axis=1)
    hm = jax.nn.relu(dot(comb, Wp1[...]) + bp1[...])
    o8 = dot(hm, Wp2[...])
    out_ref[...] = jax.nn.sigmoid(o8[:, 0:1] + bp2[0, 0])


def _tc_main(qt, ztt, weights, bn=1024):
    nb = NPAD // bn
    full = lambda shape: pl.BlockSpec(shape, lambda i: tuple(0 for _ in shape))
    in_specs = [
        pl.BlockSpec((NCORE, NCOLS, bn), lambda i: (0, 0, i)),
        pl.BlockSpec((16, bn), lambda i: (0, i)),
    ] + [full(w.shape) for w in weights]
    return pl.pallas_call(
        _main_body,
        grid=(nb,),
        in_specs=in_specs,
        out_specs=pl.BlockSpec((bn, 1), lambda i: (i, 0)),
        out_shape=jax.ShapeDtypeStruct((NPAD, 1), jnp.float32),
    )(qt, ztt, *weights)


def kernel(snapshots, edge_index, W_embed, b_embed, W_gcn, b_gcn, W_ih, W_hh,
           b_ih, b_hh, W_in_proj, b_in_proj, W_out_proj, b_out_proj, ln_g,
           ln_b, W_p1, b_p1, W_p2, b_p2):
    f32 = jnp.float32
    src = edge_index[0]
    dst = edge_index[1]
    sink = NN + (jnp.arange(EPAD - EE, dtype=jnp.int32) % (NPAD - NN))
    src_p = jnp.concatenate([src, sink])
    dst_p = jnp.concatenate([dst, sink])

    sc_deg, sc_gs = _sc_kernels()
    degp = sc_deg(dst_p)
    s16 = jnp.zeros((16, NPAD), f32).at[0:TT, :NN].set(snapshots)
    ztt = _tc_prep(s16, degp[0:1], degp[1:2])
    qt = sc_gs(src_p, dst_p, ztt)

    r1 = lambda v: v.reshape(1, -1)
    W_p2_pad = jnp.zeros((8, HH), f32).at[0:1, :].set(W_p2)
    weights = (W_gcn, W_ih, W_hh, W_in_proj, W_out_proj, W_p1, W_p2_pad,
               r1(W_embed), r1(b_embed), r1(b_gcn), r1(b_ih), r1(b_hh),
               r1(b_in_proj), r1(b_out_proj), r1(ln_g), r1(ln_b), r1(b_p1),
               r1(b_p2))
    out = _tc_main(qt, ztt, weights)
    return out[:NN, 0]


# speedup vs baseline: 1.1711x; 1.1064x over previous
import functools

import jax
import jax.numpy as jnp
from jax import lax
from jax.experimental import pallas as pl
from jax.experimental.pallas import tpu as pltpu
from jax.experimental.pallas import tpu_sc as plsc

NN = 10000
NPAD = 10240
TT = 12
HH = 128
EE = 320000
NHEADS = 4
DH = HH // NHEADS

NSUB = 16
NCORE = 2
SLAB = NPAD // NSUB
CH = 128
CPW = 79
EPW = CPW * CH
EPAD = NCORE * NSUB * EPW
NCOLS = TT + 1
NCG = 4
NEQ = 4
EPQ = EPAD // NCORE // NEQ
QCH = EPQ // NEQ
QSLAB = NPAD // NEQ


def _sc_deg_body(dst_hbm, out_hbm, dstbuf, acc, redbuf, shared):
    c = lax.axis_index("c")
    s = lax.axis_index("s")
    w = c * NSUB + s
    ones16 = jnp.ones((16,), jnp.float32)
    zero16 = jnp.zeros((16,), jnp.float32)

    pltpu.sync_copy(dst_hbm.at[pl.ds(w * EPW, EPW)], dstbuf)

    def zb(i, cr):
        acc[pl.ds(i * 16, 16)] = zero16
        return cr

    lax.fori_loop(0, NPAD // 16, zb, 0)

    def eb(o, cr):
        for u2 in range(CH // 16):
            di = dstbuf[pl.ds(o * CH + u2 * 16, 16)]
            plsc.addupdate_scatter(acc, [di], ones16)
        return cr

    lax.fori_loop(0, CPW, eb, 0)

    pltpu.sync_copy(acc, shared.at[s])
    plsc.subcore_barrier()
    base = s * SLAB
    for half in range(2):
        pltpu.sync_copy(shared.at[pl.ds(half * 8, 8), pl.ds(base, SLAB)],
                        redbuf)

        def rb(g, cr, _half=half):
            v = redbuf[0, pl.ds(g * 16, 16)]
            for rr in range(1, 8):
                v = v + redbuf[rr, pl.ds(g * 16, 16)]
            o = base + g * 16
            if _half:
                acc[pl.ds(o, 16)] = acc[pl.ds(o, 16)] + v
            else:
                acc[pl.ds(o, 16)] = v
            return cr

        lax.fori_loop(0, SLAB // 16, rb, 0)
    pltpu.sync_copy(acc.at[pl.ds(base, SLAB)], out_hbm.at[c, pl.ds(base, SLAB)])


def _sc_gs_body(src_hbm, dst_hbm, ztt_hbm, out_hbm, srcbuf, dstbuf,
                tb0, tb1, tb2, tb3, ac0, ac1, ac2, ac3,
                redbuf, shared):
    c = lax.axis_index("c")
    s = lax.axis_index("s")
    cg = s // NEQ
    eq = s % NEQ
    zero16 = jnp.zeros((16,), jnp.float32)
    tables = (tb0, tb1, tb2, tb3)
    accs = (ac0, ac1, ac2, ac3)

    def zb(i, cr):
        for ci in range(NCG):
            accs[ci][pl.ds(i * 16, 16)] = zero16
        return cr

    lax.fori_loop(0, NPAD // 16, zb, 0)
    for ci in range(NCG):
        pltpu.sync_copy(ztt_hbm.at[cg * NCG + ci], tables[ci])

    ebase = c * (EPAD // NCORE) + eq * EPQ
    for q in range(NEQ):
        pltpu.sync_copy(src_hbm.at[pl.ds(ebase + q * QCH, QCH)], srcbuf)
        pltpu.sync_copy(dst_hbm.at[pl.ds(ebase + q * QCH, QCH)], dstbuf)

        def eb(o, cr):
            for u2 in range(CH // 16):
                off = o * CH + u2 * 16
                si = srcbuf[pl.ds(off, 16)]
                di = dstbuf[pl.ds(off, 16)]
                for ci in range(NCG):
                    vals = plsc.load_gather(tables[ci], [si])
                    plsc.addupdate_scatter(accs[ci], [di], vals)
            return cr

        lax.fori_loop(0, QCH // CH, eb, 0)

    rbase = eq * QSLAB
    for ci in range(NCG):
        pltpu.sync_copy(accs[ci], shared.at[s])
        plsc.subcore_barrier()
        pltpu.sync_copy(
            shared.at[pl.ds(cg * NEQ, NEQ), pl.ds(rbase, QSLAB)], redbuf)

        def rb(g, cr, _ci=ci):
            v = redbuf[0, pl.ds(g * 16, 16)]
            for rr in range(1, NEQ):
                v = v + redbuf[rr, pl.ds(g * 16, 16)]
            accs[_ci][pl.ds(rbase + g * 16, 16)] = v
            return cr

        lax.fori_loop(0, QSLAB // 16, rb, 0)
        pltpu.sync_copy(accs[ci].at[pl.ds(rbase, QSLAB)],
                        out_hbm.at[c, cg * NCG + ci, pl.ds(rbase, QSLAB)])
        plsc.subcore_barrier()


@functools.cache
def _sc_kernels():
    mesh = plsc.VectorSubcoreMesh(core_axis_name="c", subcore_axis_name="s",
                                  num_cores=NCORE, num_subcores=NSUB)
    params = pltpu.CompilerParams(needs_layout_passes=False,
                                  use_tc_tiling_on_sc=False)
    sc_deg = pl.kernel(
        _sc_deg_body,
        out_type=jax.ShapeDtypeStruct((NCORE, NPAD), jnp.float32),
        mesh=mesh,
        compiler_params=params,
        scratch_types=[
            pltpu.VMEM((EPW,), jnp.int32),
            pltpu.VMEM((NPAD,), jnp.float32),
            pltpu.VMEM((8, SLAB), jnp.float32),
            pltpu.VMEM_SHARED((NSUB, NPAD), jnp.float32),
        ],
    )
    sc_gs = pl.kernel(
        _sc_gs_body,
        out_type=jax.ShapeDtypeStruct((NCORE, 16, NPAD), jnp.float32),
        mesh=mesh,
        compiler_params=params,
        scratch_types=(
            [pltpu.VMEM((QCH,), jnp.int32)] * 2
            + [pltpu.VMEM((NPAD,), jnp.float32)] * (2 * NCG)
            + [pltpu.VMEM((NEQ, QSLAB), jnp.float32),
               pltpu.VMEM_SHARED((NSUB, NPAD), jnp.float32)]
        ),
    )
    return sc_deg, sc_gs


def _prep_body(s16_ref, d0_ref, d1_ref, ztt_ref):
    deg = d0_ref[...] + d1_ref[...] + 1.0
    dinv = lax.rsqrt(deg)
    sel = (lax.broadcasted_iota(jnp.int32, (16, 1), 0) == TT).astype(
        jnp.float32)
    mm = lambda x, w_: lax.dot_general(
        x, w_, (((1,), (0,)), ((), ())), preferred_element_type=jnp.float32)
    ztt_ref[...] = s16_ref[...] * dinv + mm(sel, dinv)


def _tc_prep(s16, d0, d1, bn=1024):
    return pl.pallas_call(
        _prep_body,
        grid=(NPAD // bn,),
        in_specs=[
            pl.BlockSpec((16, bn), lambda i: (0, i)),
            pl.BlockSpec((1, bn), lambda i: (0, i)),
            pl.BlockSpec((1, bn), lambda i: (0, i)),
        ],
        out_specs=pl.BlockSpec((16, bn), lambda i: (0, i)),
        out_shape=jax.ShapeDtypeStruct((16, NPAD), jnp.float32),
    )(s16, d0, d1)


def _main_body(qt_ref, ztt_ref, Wgcn, Wih, Whh, Wip, Wop, Wp1, Wp2,
               we, be, bg, bih, bhh, bip, bop, lng, lnb, bp1, bp2, out_ref):
    f32 = jnp.float32
    dot = lambda x, w_: lax.dot_general(
        x, w_, (((1,), (1,)), ((), ())), preferred_element_type=f32)
    mm = lambda x, w_: lax.dot_general(
        x, w_, (((1,), (0,)), ((), ())), preferred_element_type=f32)
    ones16 = jnp.ones((1, 16), f32)
    bc16 = lambda col: mm(col, ones16)

    zttb = ztt_ref[...]
    bn = zttb.shape[1]
    dinv_row = zttb[12:13, :]
    qpad = qt_ref[0] + qt_ref[1]
    acols = jnp.transpose(dinv_row * (qpad + zttb), (1, 0))
    dd = acols[:, 12:13]
    a_last = acols[:, 11:12]
    lane16 = lax.broadcasted_iota(jnp.int32, (1, 16), 1)
    tmask = (lane16 < TT).astype(f32)
    a16 = acols * tmask

    we_, be_, bg_ = we[...], be[...], bg[...]
    u = dot(we_, Wgcn[...])
    cvec = dot(be_, Wgcn[...])
    ui = dot(u, Wih[...])
    ci = dot(cvec, Wih[...])
    bi0 = dot(bg_, Wih[...]) + bih[...]

    h = jnp.zeros((bn, HH), f32)
    gi_base = mm(dd, ci) + bi0
    bhh_r = bhh[...]
    for t in range(TT):
        gi = mm(acols[:, t:t + 1], ui) + gi_base
        gh = dot(h, Whh[...]) + bhh_r
        r = jax.nn.sigmoid(gi[:, 0:HH] + gh[:, 0:HH])
        z = jax.nn.sigmoid(gi[:, HH:2 * HH] + gh[:, HH:2 * HH])
        ng = jnp.tanh(gi[:, 2 * HH:] + r * gh[:, 2 * HH:])
        h = (1.0 - z) * ng + z * h

    pu = dot(u, Wip[...])
    pc = dot(cvec, Wip[...])
    pb = dot(bg_, Wip[...]) + bip[...]
    scl = 1.0 / (DH ** 0.5)

    puv = pu[:, 2 * HH:]
    pcv = pc[:, 2 * HH:]
    pbv = pb[:, 2 * HH:]
    cv = dot(pcv, Wop[...])
    bv = dot(pbv, Wop[...]) + bop[...]

    y = mm(a_last, u) + mm(dd, cvec + cv) + (bg_ + bv)
    negbig = (lane16 >= TT).astype(f32) * (-1e30)
    for hd in range(NHEADS):
        o = hd * DH
        puq = pu[:, o:o + DH]; puk = pu[:, HH + o:HH + o + DH]
        pcq = pc[:, o:o + DH]; pck = pc[:, HH + o:HH + o + DH]
        pbq = pb[:, o:o + DH]; pbk = pb[:, HH + o:HH + o + DH]
        d11 = lambda x, yv: jnp.sum(x * yv)
        c_qk = d11(puq, puk)
        c_qB = d11(puq, pck); c_qb = d11(puq, pbk)
        c_Ak = d11(pcq, puk); c_ak = d11(pbq, puk)
        c_AA = d11(pcq, pck)
        c_Ab = d11(pcq, pbk) + d11(pbq, pck)
        c_bb = d11(pbq, pbk)
        f1 = (a_last * c_qk + dd * c_Ak + c_ak) * scl
        f0 = (a_last * (dd * c_qB + c_qb)
              + dd * dd * c_AA + dd * c_Ab + c_bb) * scl
        scores = a16 * bc16(f1) + bc16(f0) + negbig
        m = jnp.max(scores, axis=1, keepdims=True)
        e = jnp.exp(scores - bc16(m))
        recip = 1.0 / jnp.sum(e, axis=1, keepdims=True)
        att = e * bc16(recip)
        wh = jnp.sum(att * a16, axis=1, keepdims=True)
        eh = dot(puv[:, o:o + DH], Wop[:, o:o + DH])
        y = y + mm(wh, eh)

    ones128 = jnp.ones((1, HH), f32)
    mu = jnp.mean(y, axis=1, keepdims=True)
    yc = y - mm(mu, ones128)
    var = jnp.mean(yc * yc, axis=1, keepdims=True)
    irs = mm(lax.rsqrt(var + 1e-5), ones128)
    gt = yc * irs * lng[...] + lnb[...]

    comb = jnp.concatenate([h, gt], axis=1)
    hm = jax.nn.relu(dot(comb, Wp1[...]) + bp1[...])
    o8 = dot(hm, Wp2[...])
    out_ref[...] = jax.nn.sigmoid(o8[:, 0:1] + bp2[0, 0])


def _tc_main(qt, ztt, weights, bn=1024):
    nb = NPAD // bn
    full = lambda shape: pl.BlockSpec(shape, lambda i: tuple(0 for _ in shape))
    in_specs = [
        pl.BlockSpec((NCORE, 16, bn), lambda i: (0, 0, i)),
        pl.BlockSpec((16, bn), lambda i: (0, i)),
    ] + [full(w.shape) for w in weights]
    return pl.pallas_call(
        _main_body,
        grid=(nb,),
        in_specs=in_specs,
        out_specs=pl.BlockSpec((bn, 1), lambda i: (i, 0)),
        out_shape=jax.ShapeDtypeStruct((NPAD, 1), jnp.float32),
    )(qt, ztt, *weights)


def kernel(snapshots, edge_index, W_embed, b_embed, W_gcn, b_gcn, W_ih, W_hh,
           b_ih, b_hh, W_in_proj, b_in_proj, W_out_proj, b_out_proj, ln_g,
           ln_b, W_p1, b_p1, W_p2, b_p2):
    f32 = jnp.float32
    src = edge_index[0]
    dst = edge_index[1]
    sink = NN + (jnp.arange(EPAD - EE, dtype=jnp.int32) % (NPAD - NN))
    src_p = jnp.concatenate([src, sink])
    dst_p = jnp.concatenate([dst, sink])

    sc_deg, sc_gs = _sc_kernels()
    degp = sc_deg(dst_p)
    s16 = jnp.zeros((16, NPAD), f32).at[0:TT, :NN].set(snapshots)
    ztt = _tc_prep(s16, degp[0:1], degp[1:2])
    qt = sc_gs(src_p, dst_p, ztt)

    r1 = lambda v: v.reshape(1, -1)
    W_p2_pad = jnp.zeros((8, HH), f32).at[0:1, :].set(W_p2)
    weights = (W_gcn, W_ih, W_hh, W_in_proj, W_out_proj, W_p1, W_p2_pad,
               r1(W_embed), r1(b_embed), r1(b_gcn), r1(b_ih), r1(b_hh),
               r1(b_in_proj), r1(b_out_proj), r1(ln_g), r1(ln_b), r1(b_p1),
               r1(b_p2))
    out = _tc_main(qt, ztt, weights)
    return out[:NN, 0]


# speedup vs baseline: 1.1817x; 1.0091x over previous
import functools

import jax
import jax.numpy as jnp
from jax import lax
from jax.experimental import pallas as pl
from jax.experimental.pallas import tpu as pltpu
from jax.experimental.pallas import tpu_sc as plsc

NN = 10000
NPAD = 10240
TT = 12
HH = 128
EE = 320000
NHEADS = 4
DH = HH // NHEADS

NSUB = 16
NCORE = 2
SLAB = NPAD // NSUB
CH = 128
CPW = 79
EPW = CPW * CH
EPAD = NCORE * NSUB * EPW
NCOLS = TT + 1
NCG = 4
NEQ = 4
EPQ = EPAD // NCORE // NEQ
QCH = EPQ // NEQ
QSLAB = NPAD // NEQ


def _sc_all_body(src_hbm, dst_hbm, s16_hbm, ztt_hbm, qt_hbm, srcbuf, dstbuf,
                 tb0, tb1, tb2, tb3, ac0, ac1, ac2, ac3,
                 redg, redd, s16buf, shared):
    c = lax.axis_index("c")
    s = lax.axis_index("s")
    cg = s // NEQ
    eq = s % NEQ
    zero16 = jnp.zeros((16,), jnp.float32)
    ones16 = jnp.ones((16,), jnp.float32)
    tables = (tb0, tb1, tb2, tb3)
    accs = (ac0, ac1, ac2, ac3)
    base = s * SLAB

    def zb0(i, cr):
        ac0[pl.ds(i * 16, 16)] = zero16
        return cr

    lax.fori_loop(0, NPAD // 16, zb0, 0)
    for q in range(2):
        pltpu.sync_copy(
            dst_hbm.at[pl.ds((s * 2 + q) * QCH, QCH)], dstbuf)

        def db(o, cr):
            for u2 in range(CH // 16):
                di = dstbuf[pl.ds(o * CH + u2 * 16, 16)]
                plsc.addupdate_scatter(ac0, [di], ones16)
            return cr

        lax.fori_loop(0, QCH // CH, db, 0)

    pltpu.sync_copy(ac0, shared.at[s])
    plsc.subcore_barrier()
    for half in range(2):
        pltpu.sync_copy(shared.at[pl.ds(half * 8, 8), pl.ds(base, SLAB)],
                        redd)

        def rb0(g, cr, _half=half):
            v = redd[0, pl.ds(g * 16, 16)]
            for rr in range(1, 8):
                v = v + redd[rr, pl.ds(g * 16, 16)]
            o = base + g * 16
            if _half:
                ac0[pl.ds(o, 16)] = ac0[pl.ds(o, 16)] + v + 1.0
            else:
                ac0[pl.ds(o, 16)] = v
            return cr

        lax.fori_loop(0, SLAB // 16, rb0, 0)

    pltpu.sync_copy(s16_hbm.at[:, pl.ds(base, SLAB)], s16buf)

    def pr(g, cr):
        x = ac0[pl.ds(base + g * 16, 16)]
        i32 = plsc.bitcast(x, jnp.int32)
        y = plsc.bitcast(jnp.int32(0x5F3759DF) - (i32 >> 1), jnp.float32)
        for _ in range(3):
            y = y * (1.5 - 0.5 * x * y * y)
        lo = g * 16
        for t in range(16):
            if t == TT:
                s16buf[t, pl.ds(lo, 16)] = y
            else:
                s16buf[t, pl.ds(lo, 16)] = s16buf[t, pl.ds(lo, 16)] * y
        return cr

    lax.fori_loop(0, SLAB // 16, pr, 0)
    pltpu.sync_copy(s16buf, shared.at[:, pl.ds(base, SLAB)])

    @pl.when(c == 0)
    def _():
        pltpu.sync_copy(s16buf, ztt_hbm.at[:, pl.ds(base, SLAB)])

    plsc.subcore_barrier()
    for ci in range(NCG):
        pltpu.sync_copy(shared.at[cg * NCG + ci], tables[ci])
    plsc.subcore_barrier()

    def zb(i, cr):
        for ci in range(NCG):
            accs[ci][pl.ds(i * 16, 16)] = zero16
        return cr

    lax.fori_loop(0, NPAD // 16, zb, 0)

    ebase = c * (EPAD // NCORE) + eq * EPQ
    for q in range(NEQ):
        pltpu.sync_copy(src_hbm.at[pl.ds(ebase + q * QCH, QCH)], srcbuf)
        pltpu.sync_copy(dst_hbm.at[pl.ds(ebase + q * QCH, QCH)], dstbuf)

        def eb(o, cr):
            for u2 in range(CH // 16):
                off = o * CH + u2 * 16
                si = srcbuf[pl.ds(off, 16)]
                di = dstbuf[pl.ds(off, 16)]
                for ci in range(NCG):
                    vals = plsc.load_gather(tables[ci], [si])
                    plsc.addupdate_scatter(accs[ci], [di], vals)
            return cr

        lax.fori_loop(0, QCH // CH, eb, 0)

    rbase = eq * QSLAB
    for ci in range(NCG):
        pltpu.sync_copy(accs[ci], shared.at[s])
        plsc.subcore_barrier()
        for sub in range(NEQ):
            sb = rbase + sub * SLAB
            pltpu.sync_copy(
                shared.at[pl.ds(cg * NEQ, NEQ), pl.ds(sb, SLAB)], redg)

            def rb(g, cr, _ci=ci, _sb=sb):
                v = redg[0, pl.ds(g * 16, 16)]
                for rr in range(1, NEQ):
                    v = v + redg[rr, pl.ds(g * 16, 16)]
                accs[_ci][pl.ds(_sb + g * 16, 16)] = v
                return cr

            lax.fori_loop(0, SLAB // 16, rb, 0)
        pltpu.sync_copy(accs[ci].at[pl.ds(rbase, QSLAB)],
                        qt_hbm.at[c, cg * NCG + ci, pl.ds(rbase, QSLAB)])
        plsc.subcore_barrier()


@functools.cache
def _sc_kernels():
    mesh = plsc.VectorSubcoreMesh(core_axis_name="c", subcore_axis_name="s",
                                  num_cores=NCORE, num_subcores=NSUB)
    params = pltpu.CompilerParams(needs_layout_passes=False,
                                  use_tc_tiling_on_sc=False)
    sc_all = pl.kernel(
        _sc_all_body,
        out_type=(jax.ShapeDtypeStruct((16, NPAD), jnp.float32),
                  jax.ShapeDtypeStruct((NCORE, 16, NPAD), jnp.float32)),
        mesh=mesh,
        compiler_params=params,
        scratch_types=(
            [pltpu.VMEM((QCH,), jnp.int32)] * 2
            + [pltpu.VMEM((NPAD,), jnp.float32)] * (2 * NCG)
            + [pltpu.VMEM((NEQ, SLAB), jnp.float32),
               pltpu.VMEM((8, SLAB), jnp.float32),
               pltpu.VMEM((16, SLAB), jnp.float32),
               pltpu.VMEM_SHARED((NSUB, NPAD), jnp.float32)]
        ),
    )
    return sc_all


def _main_body(qt_ref, ztt_ref, Wgcn, Wih, Whh, Wip, Wop, Wp1, Wp2,
               we, be, bg, bih, bhh, bip, bop, lng, lnb, bp1, bp2, out_ref):
    f32 = jnp.float32
    dot = lambda x, w_: lax.dot_general(
        x, w_, (((1,), (1,)), ((), ())), preferred_element_type=f32)
    mm = lambda x, w_: lax.dot_general(
        x, w_, (((1,), (0,)), ((), ())), preferred_element_type=f32)
    ones16 = jnp.ones((1, 16), f32)
    bc16 = lambda col: mm(col, ones16)

    zttb = ztt_ref[...]
    bn = zttb.shape[1]
    dinv_row = zttb[12:13, :]
    qpad = qt_ref[0] + qt_ref[1]
    acols = jnp.transpose(dinv_row * (qpad + zttb), (1, 0))
    dd = acols[:, 12:13]
    a_last = acols[:, 11:12]
    lane16 = lax.broadcasted_iota(jnp.int32, (1, 16), 1)
    tmask = (lane16 < TT).astype(f32)
    a16 = acols * tmask

    we_, be_, bg_ = we[...], be[...], bg[...]
    u = dot(we_, Wgcn[...])
    cvec = dot(be_, Wgcn[...])
    ui = dot(u, Wih[...])
    ci = dot(cvec, Wih[...])
    bi0 = dot(bg_, Wih[...]) + bih[...]

    h = jnp.zeros((bn, HH), f32)
    gi_base = mm(dd, ci) + bi0
    bhh_r = bhh[...]
    for t in range(TT):
        gi = mm(acols[:, t:t + 1], ui) + gi_base
        gh = dot(h, Whh[...]) + bhh_r
        r = jax.nn.sigmoid(gi[:, 0:HH] + gh[:, 0:HH])
        z = jax.nn.sigmoid(gi[:, HH:2 * HH] + gh[:, HH:2 * HH])
        ng = jnp.tanh(gi[:, 2 * HH:] + r * gh[:, 2 * HH:])
        h = (1.0 - z) * ng + z * h

    pu = dot(u, Wip[...])
    pc = dot(cvec, Wip[...])
    pb = dot(bg_, Wip[...]) + bip[...]
    scl = 1.0 / (DH ** 0.5)

    puv = pu[:, 2 * HH:]
    pcv = pc[:, 2 * HH:]
    pbv = pb[:, 2 * HH:]
    cv = dot(pcv, Wop[...])
    bv = dot(pbv, Wop[...]) + bop[...]

    y = mm(a_last, u) + mm(dd, cvec + cv) + (bg_ + bv)
    negbig = (lane16 >= TT).astype(f32) * (-1e30)
    for hd in range(NHEADS):
        o = hd * DH
        puq = pu[:, o:o + DH]; puk = pu[:, HH + o:HH + o + DH]
        pcq = pc[:, o:o + DH]; pck = pc[:, HH + o:HH + o + DH]
        pbq = pb[:, o:o + DH]; pbk = pb[:, HH + o:HH + o + DH]
        d11 = lambda x, yv: jnp.sum(x * yv)
        c_qk = d11(puq, puk)
        c_qB = d11(puq, pck); c_qb = d11(puq, pbk)
        c_Ak = d11(pcq, puk); c_ak = d11(pbq, puk)
        c_AA = d11(pcq, pck)
        c_Ab = d11(pcq, pbk) + d11(pbq, pck)
        c_bb = d11(pbq, pbk)
        f1 = (a_last * c_qk + dd * c_Ak + c_ak) * scl
        f0 = (a_last * (dd * c_qB + c_qb)
              + dd * dd * c_AA + dd * c_Ab + c_bb) * scl
        scores = a16 * bc16(f1) + bc16(f0) + negbig
        m = jnp.max(scores, axis=1, keepdims=True)
        e = jnp.exp(scores - bc16(m))
        recip = 1.0 / jnp.sum(e, axis=1, keepdims=True)
        att = e * bc16(recip)
        wh = jnp.sum(att * a16, axis=1, keepdims=True)
        eh = dot(puv[:, o:o + DH], Wop[:, o:o + DH])
        y = y + mm(wh, eh)

    ones128 = jnp.ones((1, HH), f32)
    mu = jnp.mean(y, axis=1, keepdims=True)
    yc = y - mm(mu, ones128)
    var = jnp.mean(yc * yc, axis=1, keepdims=True)
    irs = mm(lax.rsqrt(var + 1e-5), ones128)
    gt = yc * irs * lng[...] + lnb[...]

    comb = jnp.concatenate([h, gt], axis=1)
    hm = jax.nn.relu(dot(comb, Wp1[...]) + bp1[...])
    o8 = dot(hm, Wp2[...])
    out_ref[...] = jax.nn.sigmoid(o8[:, 0:1] + bp2[0, 0])


def _tc_main(qt, ztt, weights, bn=1024):
    nb = NPAD // bn
    full = lambda shape: pl.BlockSpec(shape, lambda i: tuple(0 for _ in shape))
    in_specs = [
        pl.BlockSpec((NCORE, 16, bn), lambda i: (0, 0, i)),
        pl.BlockSpec((16, bn), lambda i: (0, i)),
    ] + [full(w.shape) for w in weights]
    return pl.pallas_call(
        _main_body,
        grid=(nb,),
        in_specs=in_specs,
        out_specs=pl.BlockSpec((bn, 1), lambda i: (i, 0)),
        out_shape=jax.ShapeDtypeStruct((NPAD, 1), jnp.float32),
    )(qt, ztt, *weights)


def kernel(snapshots, edge_index, W_embed, b_embed, W_gcn, b_gcn, W_ih, W_hh,
           b_ih, b_hh, W_in_proj, b_in_proj, W_out_proj, b_out_proj, ln_g,
           ln_b, W_p1, b_p1, W_p2, b_p2):
    f32 = jnp.float32
    src = edge_index[0]
    dst = edge_index[1]
    sink = NN + (jnp.arange(EPAD - EE, dtype=jnp.int32) % (NPAD - NN))
    src_p = jnp.concatenate([src, sink])
    dst_p = jnp.concatenate([dst, sink])

    sc_all = _sc_kernels()
    s16 = jnp.zeros((16, NPAD), f32).at[0:TT, :NN].set(snapshots)
    ztt, qt = sc_all(src_p, dst_p, s16)

    r1 = lambda v: v.reshape(1, -1)
    W_p2_pad = jnp.zeros((8, HH), f32).at[0:1, :].set(W_p2)
    weights = (W_gcn, W_ih, W_hh, W_in_proj, W_out_proj, W_p1, W_p2_pad,
               r1(W_embed), r1(b_embed), r1(b_gcn), r1(b_ih), r1(b_hh),
               r1(b_in_proj), r1(b_out_proj), r1(ln_g), r1(ln_b), r1(b_p1),
               r1(b_p2))
    out = _tc_main(qt, ztt, weights)
    return out[:NN, 0]


# speedup vs baseline: 1.1833x; 1.0013x over previous
import functools

import jax
import jax.numpy as jnp
from jax import lax
from jax.experimental import pallas as pl
from jax.experimental.pallas import tpu as pltpu
from jax.experimental.pallas import tpu_sc as plsc

NN = 10000
NPAD = 10240
TT = 12
HH = 128
EE = 320000
NHEADS = 4
DH = HH // NHEADS

NSUB = 16
NCORE = 2
SLAB = NPAD // NSUB
CH = 128
EPAD = NCORE * NSUB * 79 * CH
NCG = 4
NEQ = 4
EPQ = EPAD // NCORE // NEQ
QCH = EPQ // NEQ
QSLAB = NPAD // NEQ


def _sc_all_body(src_hbm, dst_hbm, s16_hbm, ztt_hbm, qt_hbm, srcbuf, dstbuf,
                 tb0, tb1, tb2, tb3, ac0, ac1, ac2, ac3,
                 redg, redd, s16buf, shared):
    c = lax.axis_index("c")
    s = lax.axis_index("s")
    cg = s // NEQ
    eq = s % NEQ
    zero16 = jnp.zeros((16,), jnp.float32)
    ones16 = jnp.ones((16,), jnp.float32)
    tables = (tb0, tb1, tb2, tb3)
    accs = (ac0, ac1, ac2, ac3)
    base = s * SLAB

    def zb0(i, cr):
        ac0[pl.ds(i * 16, 16)] = zero16
        return cr

    lax.fori_loop(0, NPAD // 16, zb0, 0)
    for q in range(2):
        pltpu.sync_copy(
            dst_hbm.at[pl.ds((s * 2 + q) * QCH, QCH)], dstbuf)

        def db(o, cr):
            for u2 in range(CH // 16):
                di = dstbuf[pl.ds(o * CH + u2 * 16, 16)]
                plsc.addupdate_scatter(ac0, [di], ones16)
            return cr

        lax.fori_loop(0, QCH // CH, db, 0)

    pltpu.sync_copy(ac0, shared.at[s])
    plsc.subcore_barrier()
    for half in range(2):
        pltpu.sync_copy(shared.at[pl.ds(half * 8, 8), pl.ds(base, SLAB)],
                        redd)

        def rb0(g, cr, _half=half):
            v = redd[0, pl.ds(g * 16, 16)]
            for rr in range(1, 8):
                v = v + redd[rr, pl.ds(g * 16, 16)]
            o = base + g * 16
            if _half:
                ac0[pl.ds(o, 16)] = ac0[pl.ds(o, 16)] + v + 1.0
            else:
                ac0[pl.ds(o, 16)] = v
            return cr

        lax.fori_loop(0, SLAB // 16, rb0, 0)

    pltpu.sync_copy(s16_hbm.at[:, pl.ds(base, SLAB)], s16buf)

    def pr(g, cr):
        x = ac0[pl.ds(base + g * 16, 16)]
        i32 = plsc.bitcast(x, jnp.int32)
        y = plsc.bitcast(jnp.int32(0x5F3759DF) - (i32 >> 1), jnp.float32)
        for _ in range(3):
            y = y * (1.5 - 0.5 * x * y * y)
        lo = g * 16
        for t in range(16):
            if t == TT:
                s16buf[t, pl.ds(lo, 16)] = y
            else:
                s16buf[t, pl.ds(lo, 16)] = s16buf[t, pl.ds(lo, 16)] * y
        return cr

    lax.fori_loop(0, SLAB // 16, pr, 0)
    pltpu.sync_copy(s16buf, shared.at[:, pl.ds(base, SLAB)])

    @pl.when(c == 0)
    def _():
        pltpu.sync_copy(s16buf, ztt_hbm.at[:, pl.ds(base, SLAB)])

    plsc.subcore_barrier()
    for ci in range(NCG):
        pltpu.sync_copy(shared.at[cg * NCG + ci], tables[ci])
    plsc.subcore_barrier()

    def zb(i, cr):
        for ci in range(NCG):
            accs[ci][pl.ds(i * 16, 16)] = zero16
        return cr

    lax.fori_loop(0, NPAD // 16, zb, 0)

    ebase = c * (EPAD // NCORE) + eq * EPQ
    for q in range(NEQ):
        pltpu.sync_copy(src_hbm.at[pl.ds(ebase + q * QCH, QCH)], srcbuf)
        pltpu.sync_copy(dst_hbm.at[pl.ds(ebase + q * QCH, QCH)], dstbuf)

        def eb(o, cr):
            for u2 in range(CH // 16):
                off = o * CH + u2 * 16
                si = srcbuf[pl.ds(off, 16)]
                di = dstbuf[pl.ds(off, 16)]
                for ci in range(NCG):
                    vals = plsc.load_gather(tables[ci], [si])
                    plsc.addupdate_scatter(accs[ci], [di], vals)
            return cr

        lax.fori_loop(0, QCH // CH, eb, 0)

    rbase = eq * QSLAB
    for ci in range(NCG):
        pltpu.sync_copy(accs[ci], shared.at[s])
        plsc.subcore_barrier()
        for sub in range(NEQ):
            sb = rbase + sub * SLAB
            pltpu.sync_copy(
                shared.at[pl.ds(cg * NEQ, NEQ), pl.ds(sb, SLAB)], redg)

            def rb(g, cr, _ci=ci, _sb=sb):
                v = redg[0, pl.ds(g * 16, 16)]
                for rr in range(1, NEQ):
                    v = v + redg[rr, pl.ds(g * 16, 16)]
                accs[_ci][pl.ds(_sb + g * 16, 16)] = v
                return cr

            lax.fori_loop(0, SLAB // 16, rb, 0)
        pltpu.sync_copy(accs[ci].at[pl.ds(rbase, QSLAB)],
                        qt_hbm.at[c, cg * NCG + ci, pl.ds(rbase, QSLAB)])
        plsc.subcore_barrier()


@functools.cache
def _sc_kernels():
    mesh = plsc.VectorSubcoreMesh(core_axis_name="c", subcore_axis_name="s",
                                  num_cores=NCORE, num_subcores=NSUB)
    params = pltpu.CompilerParams(needs_layout_passes=False,
                                  use_tc_tiling_on_sc=False)
    sc_all = pl.kernel(
        _sc_all_body,
        out_type=(jax.ShapeDtypeStruct((16, NPAD), jnp.float32),
                  jax.ShapeDtypeStruct((NCORE, 16, NPAD), jnp.float32)),
        mesh=mesh,
        compiler_params=params,
        scratch_types=(
            [pltpu.VMEM((QCH,), jnp.int32)] * 2
            + [pltpu.VMEM((NPAD,), jnp.float32)] * (2 * NCG)
            + [pltpu.VMEM((NEQ, SLAB), jnp.float32),
               pltpu.VMEM((8, SLAB), jnp.float32),
               pltpu.VMEM((16, SLAB), jnp.float32),
               pltpu.VMEM_SHARED((NSUB, NPAD), jnp.float32)]
        ),
    )
    return sc_all


def _main_body(qt_ref, ztt_ref, Wgcn, Wih, Whh, Wip, Wop, Wp1, Wp2,
               we, be, bg, bih, bhh, bip, bop, lng, lnb, bp1, bp2, out_ref):
    f32 = jnp.float32
    dot = lambda x, w_: lax.dot_general(
        x, w_, (((1,), (1,)), ((), ())), preferred_element_type=f32)
    mm = lambda x, w_: lax.dot_general(
        x, w_, (((1,), (0,)), ((), ())), preferred_element_type=f32)
    ones16 = jnp.ones((1, 16), f32)
    bc16 = lambda col: mm(col, ones16)

    zttb = ztt_ref[...]
    bn = zttb.shape[1]
    dinv_row = zttb[12:13, :]
    qpad = qt_ref[0] + qt_ref[1]
    acols = jnp.transpose(dinv_row * (qpad + zttb), (1, 0))
    dd = acols[:, 12:13]
    a_last = acols[:, 11:12]
    lane16 = lax.broadcasted_iota(jnp.int32, (1, 16), 1)
    tmask = (lane16 < TT).astype(f32)
    a16 = acols * tmask

    we_, be_, bg_ = we[...], be[...], bg[...]
    u = dot(we_, Wgcn[...])
    cvec = dot(be_, Wgcn[...])
    ui = dot(u, Wih[...])
    ci = dot(cvec, Wih[...])
    bi0 = dot(bg_, Wih[...]) + bih[...]

    h = jnp.zeros((bn, HH), f32)
    gi_base = mm(dd, ci) + bi0
    bhh_r = bhh[...]
    for t in range(TT):
        gi = mm(acols[:, t:t + 1], ui) + gi_base
        gh = dot(h, Whh[...]) + bhh_r
        r = jax.nn.sigmoid(gi[:, 0:HH] + gh[:, 0:HH])
        z = jax.nn.sigmoid(gi[:, HH:2 * HH] + gh[:, HH:2 * HH])
        ng = jnp.tanh(gi[:, 2 * HH:] + r * gh[:, 2 * HH:])
        h = (1.0 - z) * ng + z * h

    pu = dot(u, Wip[...])
    pc = dot(cvec, Wip[...])
    pb = dot(bg_, Wip[...]) + bip[...]
    scl = 1.0 / (DH ** 0.5)

    puv = pu[:, 2 * HH:]
    pcv = pc[:, 2 * HH:]
    pbv = pb[:, 2 * HH:]
    cv = dot(pcv, Wop[...])
    bv = dot(pbv, Wop[...]) + bop[...]

    y = mm(a_last, u) + mm(dd, cvec + cv) + (bg_ + bv)
    negbig = (lane16 >= TT).astype(f32) * (-1e30)
    for hd in range(NHEADS):
        o = hd * DH
        puq = pu[:, o:o + DH]; puk = pu[:, HH + o:HH + o + DH]
        pcq = pc[:, o:o + DH]; pck = pc[:, HH + o:HH + o + DH]
        pbq = pb[:, o:o + DH]; pbk = pb[:, HH + o:HH + o + DH]
        d11 = lambda x, yv: jnp.sum(x * yv)
        c_qk = d11(puq, puk)
        c_qB = d11(puq, pck); c_qb = d11(puq, pbk)
        c_Ak = d11(pcq, puk); c_ak = d11(pbq, puk)
        c_AA = d11(pcq, pck)
        c_Ab = d11(pcq, pbk) + d11(pbq, pck)
        c_bb = d11(pbq, pbk)
        f1 = (a_last * c_qk + dd * c_Ak + c_ak) * scl
        f0 = (a_last * (dd * c_qB + c_qb)
              + dd * dd * c_AA + dd * c_Ab + c_bb) * scl
        scores = a16 * bc16(f1) + bc16(f0) + negbig
        m = jnp.max(scores, axis=1, keepdims=True)
        e = jnp.exp(scores - bc16(m))
        recip = 1.0 / jnp.sum(e, axis=1, keepdims=True)
        att = e * bc16(recip)
        wh = jnp.sum(att * a16, axis=1, keepdims=True)
        eh = dot(puv[:, o:o + DH], Wop[:, o:o + DH])
        y = y + mm(wh, eh)

    ones128 = jnp.ones((1, HH), f32)
    mu = jnp.mean(y, axis=1, keepdims=True)
    yc = y - mm(mu, ones128)
    var = jnp.mean(yc * yc, axis=1, keepdims=True)
    irs = mm(lax.rsqrt(var + 1e-5), ones128)
    gt = yc * irs * lng[...] + lnb[...]

    comb = jnp.concatenate([h, gt], axis=1)
    hm = jax.nn.relu(dot(comb, Wp1[...]) + bp1[...])
    o8 = dot(hm, Wp2[...])
    out_ref[...] = jax.nn.sigmoid(o8[:, 0:1] + bp2[0, 0])


def _tc_main(qt, ztt, weights, bn=1024):
    nb = NPAD // bn
    full = lambda shape: pl.BlockSpec(shape, lambda i: tuple(0 for _ in shape))
    in_specs = [
        pl.BlockSpec((NCORE, 16, bn), lambda i: (0, 0, i)),
        pl.BlockSpec((16, bn), lambda i: (0, i)),
    ] + [full(w.shape) for w in weights]
    return pl.pallas_call(
        _main_body,
        grid=(nb,),
        in_specs=in_specs,
        out_specs=pl.BlockSpec((bn, 1), lambda i: (i, 0)),
        out_shape=jax.ShapeDtypeStruct((NPAD, 1), jnp.float32),
    )(qt, ztt, *weights)


def kernel(snapshots, edge_index, W_embed, b_embed, W_gcn, b_gcn, W_ih, W_hh,
           b_ih, b_hh, W_in_proj, b_in_proj, W_out_proj, b_out_proj, ln_g,
           ln_b, W_p1, b_p1, W_p2, b_p2):
    f32 = jnp.float32
    src = edge_index[0]
    dst = edge_index[1]
    sink = NN + (jnp.arange(EPAD - EE, dtype=jnp.int32) % (NPAD - NN))
    src_p = jnp.concatenate([src, sink])
    dst_p = jnp.concatenate([dst, sink])

    sc_all = _sc_kernels()
    s16 = jnp.zeros((16, NPAD), f32).at[0:TT, :NN].set(snapshots)
    ztt, qt = sc_all(src_p, dst_p, s16)

    r1 = lambda v: v.reshape(1, -1)
    W_p2_pad = jnp.zeros((8, HH), f32).at[0:1, :].set(W_p2)
    weights = (W_gcn, W_ih, W_hh, W_in_proj, W_out_proj, W_p1, W_p2_pad,
               r1(W_embed), r1(b_embed), r1(b_gcn), r1(b_ih), r1(b_hh),
               r1(b_in_proj), r1(b_out_proj), r1(ln_g), r1(ln_b), r1(b_p1),
               r1(b_p2))
    out = _tc_main(qt, ztt, weights)
    return out[:NN, 0]
